# initial kernel scaffold (unmeasured)
import jax
import jax.numpy as jnp
from jax import lax
from jax.experimental import pallas as pl
from jax.experimental.pallas import tpu as pltpu


def kernel(
    x,
):
    def body(*refs):
        pass

    out_shape = jax.ShapeDtypeStruct(..., jnp.float32)
    return pl.pallas_call(body, out_shape=out_shape)(...)


# baseline (device time: 50222 ns/iter reference)
import jax
import jax.numpy as jnp
from jax import lax
from jax.experimental import pallas as pl
from jax.experimental.pallas import tpu as pltpu

K = 16


def _topk_desc(xw, k):
    m_rows, n_cols = xw.shape
    col = lax.broadcasted_iota(jnp.int32, (m_rows, n_cols), 1)
    out_col = lax.broadcasted_iota(jnp.int32, (m_rows, k), 1)
    acc = jnp.zeros((m_rows, k), jnp.float32)
    neg = jnp.float32(-jnp.inf)
    for i in range(k):
        m = jnp.max(xw, axis=1, keepdims=True)
        acc = jnp.where(out_col == i, m, acc)
        first = jnp.min(
            jnp.where(xw == m, col, jnp.int32(n_cols)), axis=1, keepdims=True
        )
        xw = jnp.where(col == first, neg, xw)
    return acc


def kernel(x):
    m, n = x.shape

    def body(x_ref, out_ref, comm_ref, send_sem, recv_sem):
        my_x = lax.axis_index("x")
        my_y = lax.axis_index("y")
        my_z = lax.axis_index("z")
        partner = (my_x, 1 - my_y, my_z)

        comm_ref[0, :, :] = _topk_desc(x_ref[:, :], K)

        barrier_sem = pltpu.get_barrier_semaphore()
        pl.semaphore_signal(
            barrier_sem, inc=1, device_id=partner,
            device_id_type=pl.DeviceIdType.MESH,
        )
        pl.semaphore_wait(barrier_sem, 1)

        rdma = pltpu.make_async_remote_copy(
            src_ref=comm_ref.at[0],
            dst_ref=comm_ref.at[1],
            send_sem=send_sem,
            recv_sem=recv_sem,
            device_id=partner,
            device_id_type=pl.DeviceIdType.MESH,
        )
        rdma.start()
        rdma.wait()

        both = jnp.concatenate([comm_ref[0, :, :], comm_ref[1, :, :]], axis=1)
        out_ref[:, :] = _topk_desc(both, K)

    return pl.pallas_call(
        body,
        out_shape=jax.ShapeDtypeStruct((m, K), jnp.float32),
        in_specs=[pl.BlockSpec(memory_space=pltpu.VMEM)],
        out_specs=pl.BlockSpec(memory_space=pltpu.VMEM),
        scratch_shapes=[
            pltpu.VMEM((2, m, K), jnp.float32),
            pltpu.SemaphoreType.DMA,
            pltpu.SemaphoreType.DMA,
        ],
        compiler_params=pltpu.CompilerParams(collective_id=0),
    )(x)


# device time: 27965 ns/iter; 1.7959x vs baseline; 1.7959x over previous
import jax
import jax.numpy as jnp
from jax import lax
from jax.experimental import pallas as pl
from jax.experimental.pallas import tpu as pltpu

K = 16


def _topk_desc(xw, k):
    m_rows, n_cols = xw.shape
    col = lax.broadcasted_iota(jnp.int32, (m_rows, n_cols), 1)
    out_col = lax.broadcasted_iota(jnp.int32, (m_rows, k), 1)
    acc = jnp.zeros((m_rows, k), jnp.float32)
    neg = jnp.float32(-jnp.inf)
    for i in range(k):
        m = jnp.max(xw, axis=1, keepdims=True)
        acc = jnp.where(out_col == i, m, acc)
        first = jnp.min(
            jnp.where(xw == m, col, jnp.int32(n_cols)), axis=1, keepdims=True
        )
        xw = jnp.where(col == first, neg, xw)
    return acc


def _topk_desc_chain(x, k):
    m_rows, _ = x.shape
    out_col = lax.broadcasted_iota(jnp.int32, (m_rows, k), 1)
    neg = jnp.float32(-jnp.inf)
    v = jnp.max(x, axis=1, keepdims=True)
    acc = jnp.where(out_col == 0, v, jnp.zeros((m_rows, k), jnp.float32))
    for i in range(1, k):
        v = jnp.max(jnp.where(x < v, x, neg), axis=1, keepdims=True)
        acc = jnp.where(out_col == i, v, acc)
    return acc


def kernel(x):
    m, n = x.shape

    def body(x_ref, out_ref, comm_ref, send_sem, recv_sem):
        my_x = lax.axis_index("x")
        my_y = lax.axis_index("y")
        my_z = lax.axis_index("z")
        partner = (my_x, 1 - my_y, my_z)

        comm_ref[0, :, :] = _topk_desc_chain(x_ref[:, :], K)

        barrier_sem = pltpu.get_barrier_semaphore()
        pl.semaphore_signal(
            barrier_sem, inc=1, device_id=partner,
            device_id_type=pl.DeviceIdType.MESH,
        )
        pl.semaphore_wait(barrier_sem, 1)

        rdma = pltpu.make_async_remote_copy(
            src_ref=comm_ref.at[0],
            dst_ref=comm_ref.at[1],
            send_sem=send_sem,
            recv_sem=recv_sem,
            device_id=partner,
            device_id_type=pl.DeviceIdType.MESH,
        )
        rdma.start()
        rdma.wait()

        both = jnp.concatenate([comm_ref[0, :, :], comm_ref[1, :, :]], axis=1)
        out_ref[:, :] = _topk_desc(both, K)

    return pl.pallas_call(
        body,
        out_shape=jax.ShapeDtypeStruct((m, K), jnp.float32),
        in_specs=[pl.BlockSpec(memory_space=pltpu.VMEM)],
        out_specs=pl.BlockSpec(memory_space=pltpu.VMEM),
        scratch_shapes=[
            pltpu.VMEM((2, m, K), jnp.float32),
            pltpu.SemaphoreType.DMA,
            pltpu.SemaphoreType.DMA,
        ],
        compiler_params=pltpu.CompilerParams(collective_id=0),
    )(x)
